# baseline (speedup 1.0000x reference)
import jax
import jax.numpy as jnp
from jax import lax
from jax.experimental import pallas as pl
from jax.experimental.pallas import tpu as pltpu
from jax.experimental.pallas import tpu_sc as plsc

B = 16384
N_NUM = 13
N_CAT = 26
VOCAB = 100000
D = 32
N_TOK = 1 + N_NUM + N_CAT
NF2 = N_CAT // 2

_INFO = plsc.get_sparse_core_info()
NC = _INFO.num_cores
NS = _INFO.num_subcores
NW = NC * NS
SLAB = 8
N_SLABS = B // SLAB
SLABS_PER_W = N_SLABS // NW
HALF = NF2 * SLAB
MROW = 512


def _body(qw_hbm, mw_hbm, tbl_hbm, p_hbm, out_hbm,
          p_v, qw_v, mw_v, blk_v, tb0, tb1, sem0, sem1, sem_w):
  tbufs = (tb0, tb1)
  sems = (sem0, sem1)
  wid = lax.axis_index("s") * NC + lax.axis_index("c")
  sbase = wid * SLABS_PER_W

  pltpu.sync_copy(p_hbm, p_v)
  pltpu.sync_copy(qw_hbm.at[wid], qw_v)
  pltpu.sync_copy(mw_hbm.at[wid], mw_v)

  c32 = p_v[2 * N_NUM, :]
  for k in range(SLAB):
    blk_v[k, 0, :] = c32

  pltpu.async_copy(blk_v, out_hbm.at[pl.ds(sbase * SLAB, SLAB)], sem_w)

  def slab_body(c, _):
    bb = (sbase + c) * SLAB

    gathers = []
    for h in range(2):
      gathers.append(
          pltpu.async_copy(tbl_hbm.at[qw_v.at[c, h]], tbufs[h], sems[h]))

    pltpu.make_async_copy(
        blk_v, out_hbm.at[pl.ds(bb, SLAB)], sem_w).wait()

    for j in range(N_NUM):
      sj = lax.bitcast_convert_type(
          mw_v[c, pl.ds(256 + j * SLAB, 16)], jnp.float32)
      a32 = p_v[j, :]
      b32 = p_v[N_NUM + j, :]
      for k in range(SLAB):
        blk_v[k, 1 + j, :] = sj[k] * a32 + b32

    for h in range(2):
      gathers[h].wait()
      tb = tbufs[h]
      for fi in range(NF2):
        f = h * NF2 + fi
        rem16 = mw_v[c, pl.ds(f * SLAB, 16)]
        for k in range(SLAB):
          mm = rem16[k]
          blk_v[k, 1 + N_NUM + f, :] = tb[fi * SLAB + k, pl.ds(mm * D, D)]

    pltpu.async_copy(blk_v, out_hbm.at[pl.ds(bb, SLAB)], sem_w)
    return _

  lax.fori_loop(0, SLABS_PER_W, slab_body, None)
  pltpu.make_async_copy(
      blk_v, out_hbm.at[pl.ds(sbase * SLAB, SLAB)], sem_w).wait()


@jax.jit
def _tokenize(qw, mw, tbl, p):
  mesh = plsc.VectorSubcoreMesh(core_axis_name="c", subcore_axis_name="s")
  return pl.kernel(
      _body,
      mesh=mesh,
      out_type=jax.ShapeDtypeStruct((B, N_TOK, D), jnp.float32),
      scratch_types=[
          pltpu.VMEM((2 * N_NUM + 1, D), jnp.float32),
          pltpu.VMEM((SLABS_PER_W, 2, HALF), jnp.int32),
          pltpu.VMEM((SLABS_PER_W, MROW), jnp.int32),
          pltpu.VMEM((SLAB, N_TOK, D), jnp.float32),
          pltpu.VMEM((HALF, 128), jnp.float32),
          pltpu.VMEM((HALF, 128), jnp.float32),
          pltpu.SemaphoreType.DMA,
          pltpu.SemaphoreType.DMA,
          pltpu.SemaphoreType.DMA,
      ],
  )(qw, mw, tbl, p)


def kernel(x_num, x_cat, num_weight, num_bias, cat_tables, cls):
  r = x_cat + jnp.arange(N_CAT, dtype=jnp.int32)[None, :] * VOCAB
  qt = jnp.transpose((r >> 2).reshape(N_SLABS, SLAB, N_CAT), (0, 2, 1))
  qw = qt.reshape(NW, SLABS_PER_W, 2, HALF)
  rem = jnp.transpose((r & 3).reshape(N_SLABS, SLAB, N_CAT), (0, 2, 1))
  rem = rem.reshape(N_SLABS, N_CAT * SLAB)
  xnt = lax.bitcast_convert_type(
      jnp.transpose(x_num.reshape(N_SLABS, SLAB, N_NUM), (0, 2, 1)),
      jnp.int32).reshape(N_SLABS, N_NUM * SLAB)
  mw = jnp.concatenate(
      [rem, jnp.zeros((N_SLABS, 256 - N_CAT * SLAB), jnp.int32),
       xnt, jnp.zeros((N_SLABS, MROW - 256 - N_NUM * SLAB), jnp.int32)],
      axis=1).reshape(NW, SLABS_PER_W, MROW)
  tbl = cat_tables.reshape(N_CAT * VOCAB // 4, 4 * D)
  p = jnp.concatenate([num_weight, num_bias, cls.reshape(1, D)], axis=0)
  return _tokenize(qw, mw, tbl, p)


# speedup vs baseline: 1.0026x; 1.0026x over previous
import jax
import jax.numpy as jnp
from jax import lax
from jax.experimental import pallas as pl
from jax.experimental.pallas import tpu as pltpu
from jax.experimental.pallas import tpu_sc as plsc

B = 16384
N_NUM = 13
N_CAT = 26
VOCAB = 100000
D = 32
N_TOK = 1 + N_NUM + N_CAT
NF2 = N_CAT // 2

_INFO = plsc.get_sparse_core_info()
NC = _INFO.num_cores
NS = _INFO.num_subcores
NW = NC * NS
SLAB = 8
N_SLABS = B // SLAB
SLABS_PER_W = N_SLABS // NW
HALF = NF2 * SLAB
MROW = 512


def _body(qw_hbm, mw_hbm, tbl_hbm, p_hbm, out_hbm,
          p_v, qw_v, mw_v, blk_v, tb0, tb1, sem0, sem1, sem_w):
  tbufs = (tb0, tb1)
  sems = (sem0, sem1)
  wid = lax.axis_index("s") * NC + lax.axis_index("c")
  sbase = wid * SLABS_PER_W

  pltpu.sync_copy(p_hbm, p_v)
  pltpu.sync_copy(qw_hbm.at[wid], qw_v)
  pltpu.sync_copy(mw_hbm.at[wid], mw_v)

  c32 = p_v[2 * N_NUM, :]
  for k in range(SLAB):
    blk_v[k, 0, :] = c32

  pltpu.async_copy(blk_v, out_hbm.at[pl.ds(sbase * SLAB, SLAB)], sem_w)
  for h in range(2):
    pltpu.async_copy(tbl_hbm.at[qw_v.at[0, h]], tbufs[h], sems[h])

  def slab_body(c, _):
    bb = (sbase + c) * SLAB

    pltpu.make_async_copy(
        blk_v, out_hbm.at[pl.ds(bb, SLAB)], sem_w).wait()

    for j in range(N_NUM):
      sj = lax.bitcast_convert_type(
          mw_v[c, pl.ds(256 + j * SLAB, 16)], jnp.float32)
      a32 = p_v[j, :]
      b32 = p_v[N_NUM + j, :]
      for k in range(SLAB):
        blk_v[k, 1 + j, :] = sj[k] * a32 + b32

    for h in range(2):
      tb = tbufs[h]
      pltpu.make_async_copy(
          tbl_hbm.at[qw_v.at[c, h]], tb, sems[h]).wait()
      for fi in range(NF2):
        f = h * NF2 + fi
        rem16 = mw_v[c, pl.ds(f * SLAB, 16)]
        for k in range(SLAB):
          mm = rem16[k]
          blk_v[k, 1 + N_NUM + f, :] = tb[fi * SLAB + k, pl.ds(mm * D, D)]
      cn = jnp.minimum(c + 1, SLABS_PER_W - 1)

      @pl.when(c + 1 < SLABS_PER_W)
      def _fire(h=h, tb=tb, cn=cn):
        pltpu.async_copy(tbl_hbm.at[qw_v.at[cn, h]], tb, sems[h])

    pltpu.async_copy(blk_v, out_hbm.at[pl.ds(bb, SLAB)], sem_w)
    return _

  lax.fori_loop(0, SLABS_PER_W, slab_body, None)
  pltpu.make_async_copy(
      blk_v, out_hbm.at[pl.ds(sbase * SLAB, SLAB)], sem_w).wait()


@jax.jit
def _tokenize(qw, mw, tbl, p):
  mesh = plsc.VectorSubcoreMesh(core_axis_name="c", subcore_axis_name="s")
  return pl.kernel(
      _body,
      mesh=mesh,
      out_type=jax.ShapeDtypeStruct((B, N_TOK, D), jnp.float32),
      scratch_types=[
          pltpu.VMEM((2 * N_NUM + 1, D), jnp.float32),
          pltpu.VMEM((SLABS_PER_W, 2, HALF), jnp.int32),
          pltpu.VMEM((SLABS_PER_W, MROW), jnp.int32),
          pltpu.VMEM((SLAB, N_TOK, D), jnp.float32),
          pltpu.VMEM((HALF, 128), jnp.float32),
          pltpu.VMEM((HALF, 128), jnp.float32),
          pltpu.SemaphoreType.DMA,
          pltpu.SemaphoreType.DMA,
          pltpu.SemaphoreType.DMA,
      ],
  )(qw, mw, tbl, p)


def kernel(x_num, x_cat, num_weight, num_bias, cat_tables, cls):
  r = x_cat + jnp.arange(N_CAT, dtype=jnp.int32)[None, :] * VOCAB
  qt = jnp.transpose((r >> 2).reshape(N_SLABS, SLAB, N_CAT), (0, 2, 1))
  qw = qt.reshape(NW, SLABS_PER_W, 2, HALF)
  rem = jnp.transpose((r & 3).reshape(N_SLABS, SLAB, N_CAT), (0, 2, 1))
  rem = rem.reshape(N_SLABS, N_CAT * SLAB)
  xnt = lax.bitcast_convert_type(
      jnp.transpose(x_num.reshape(N_SLABS, SLAB, N_NUM), (0, 2, 1)),
      jnp.int32).reshape(N_SLABS, N_NUM * SLAB)
  mw = jnp.concatenate(
      [rem, jnp.zeros((N_SLABS, 256 - N_CAT * SLAB), jnp.int32),
       xnt, jnp.zeros((N_SLABS, MROW - 256 - N_NUM * SLAB), jnp.int32)],
      axis=1).reshape(NW, SLABS_PER_W, MROW)
  tbl = cat_tables.reshape(N_CAT * VOCAB // 4, 4 * D)
  p = jnp.concatenate([num_weight, num_bias, cls.reshape(1, D)], axis=0)
  return _tokenize(qw, mw, tbl, p)
